# baseline (speedup 1.0000x reference)
import jax
import jax.numpy as jnp
from jax.experimental import pallas as pl
from jax.experimental.pallas import tpu as pltpu

D_MODEL = 640
INTERMEDIATE_SIZE = 640
EXPERTS_PER_TOKEN = 4
RMS_NORM_EPS = 1e-05
SWIGLU_LIMIT = 7.0
N_EXPERTS = 16
SEQ_LEN = 32
NBUF = 4


def _moe_kernel(xdt_ref, nw_ref, gw_ref, gb_ref, m1w_hbm, m1b_ref, m2w_hbm,
                m2b_ref, out_ref, acc_s, t_s, w_s, m1buf, m2buf, m1sem, m2sem):
    e = pl.program_id(0)
    T, D, I, E, K = SEQ_LEN, D_MODEL, INTERMEDIATE_SIZE, N_EXPERTS, EXPERTS_PER_TOKEN

    @pl.when(e == 0)
    def _prologue():
        for i in range(NBUF):
            pltpu.make_async_copy(m1w_hbm.at[i], m1buf.at[i], m1sem.at[i]).start()
            pltpu.make_async_copy(m2w_hbm.at[i], m2buf.at[i], m2sem.at[i]).start()
        xt = jnp.transpose(xdt_ref[...])
        var = jnp.mean(xt * xt, axis=1, keepdims=True)
        t = xt * jax.lax.rsqrt(var + RMS_NORM_EPS) * nw_ref[...]
        t_s[...] = t
        g = jax.lax.dot_general(t, gw_ref[...], (((1,), (1,)), ((), ())),
                                preferred_element_type=jnp.float32)
        g = g + gb_ref[...]
        lane = jax.lax.broadcasted_iota(jnp.int32, (T, E), 1)
        rank = jnp.zeros((T, E), dtype=jnp.int32)
        for j in range(E):
            gj = g[:, j:j + 1]
            rank = rank + (gj > g).astype(jnp.int32)
            rank = rank + ((gj == g) & (j < lane)).astype(jnp.int32)
        sel = rank < K
        gm = jnp.where(sel, g, jnp.float32(-jnp.inf))
        mx = jnp.max(gm, axis=1, keepdims=True)
        ex = jnp.where(sel, jnp.exp(g - mx), 0.0)
        w_s[...] = ex / jnp.sum(ex, axis=1, keepdims=True)

    slot = jax.lax.rem(e, NBUF)
    pltpu.make_async_copy(m1w_hbm.at[e], m1buf.at[slot], m1sem.at[slot]).wait()
    pltpu.make_async_copy(m2w_hbm.at[e], m2buf.at[slot], m2sem.at[slot]).wait()

    t = t_s[...]
    h = jnp.dot(t, m1buf[slot], preferred_element_type=jnp.float32)
    h = h + m1b_ref[0]
    h_glu = jnp.minimum(h[:, :I], SWIGLU_LIMIT)
    h_lin = jnp.clip(h[:, I:], -SWIGLU_LIMIT, SWIGLU_LIMIT)
    act = h_glu * jax.nn.sigmoid(1.702 * h_glu) * (h_lin + 1.0)
    o = jnp.dot(act, m2buf[slot], preferred_element_type=jnp.float32)
    o = o + m2b_ref[0]

    lane_e = jax.lax.broadcasted_iota(jnp.int32, (T, E), 1)
    wcol = jnp.sum(jnp.where(lane_e == e, w_s[...], 0.0), axis=1,
                   keepdims=True)
    contrib = wcol * o

    @pl.when(e == 0)
    def _init():
        acc_s[...] = jnp.transpose(xdt_ref[...]) + contrib

    @pl.when(e != 0)
    def _acc():
        acc_s[...] += contrib

    @pl.when(e == E - 1)
    def _finish():
        out_ref[...] = jnp.transpose(acc_s[...])

    @pl.when(e + NBUF < E)
    def _prefetch():
        nxt = e + NBUF
        pltpu.make_async_copy(m1w_hbm.at[nxt], m1buf.at[slot], m1sem.at[slot]).start()
        pltpu.make_async_copy(m2w_hbm.at[nxt], m2buf.at[slot], m2sem.at[slot]).start()


@jax.jit
def kernel(x, norm_weight, gate_weight, gate_bias, mlp1_weight, mlp1_bias,
           mlp2_weight, mlp2_bias):
    T, D, I, E = SEQ_LEN, D_MODEL, INTERMEDIATE_SIZE, N_EXPERTS
    xdt = x.reshape(D, T)
    out = pl.pallas_call(
        _moe_kernel,
        grid=(E,),
        in_specs=[
            pl.BlockSpec((D, T), lambda e: (0, 0)),
            pl.BlockSpec((1, D), lambda e: (0, 0)),
            pl.BlockSpec((E, D), lambda e: (0, 0)),
            pl.BlockSpec((1, E), lambda e: (0, 0)),
            pl.BlockSpec(memory_space=pl.ANY),
            pl.BlockSpec((1, 1, 2 * I), lambda e: (e, 0, 0)),
            pl.BlockSpec(memory_space=pl.ANY),
            pl.BlockSpec((1, 1, D), lambda e: (e, 0, 0)),
        ],
        out_specs=pl.BlockSpec((D, T), lambda e: (0, 0)),
        out_shape=jax.ShapeDtypeStruct((D, T), jnp.float32),
        scratch_shapes=[
            pltpu.VMEM((T, D), jnp.float32),
            pltpu.VMEM((T, D), jnp.float32),
            pltpu.VMEM((T, E), jnp.float32),
            pltpu.VMEM((NBUF, D, 2 * I), jnp.float32),
            pltpu.VMEM((NBUF, I, D), jnp.float32),
            pltpu.SemaphoreType.DMA((NBUF,)),
            pltpu.SemaphoreType.DMA((NBUF,)),
        ],
        compiler_params=pltpu.CompilerParams(
            dimension_semantics=("arbitrary",),
        ),
    )(xdt, norm_weight.reshape(1, D), gate_weight, gate_bias.reshape(1, E),
      mlp1_weight, mlp1_bias.reshape(E, 1, 2 * I), mlp2_weight,
      mlp2_bias.reshape(E, 1, D))
    return out.reshape(1, D, 1, T)


# speedup vs baseline: 1.0750x; 1.0750x over previous
import jax
import jax.numpy as jnp
from jax.experimental import pallas as pl
from jax.experimental.pallas import tpu as pltpu

D_MODEL = 640
INTERMEDIATE_SIZE = 640
EXPERTS_PER_TOKEN = 4
RMS_NORM_EPS = 1e-05
SWIGLU_LIMIT = 7.0
N_EXPERTS = 16
SEQ_LEN = 32
NBUF = 6


def _start_slab(src, dst, sems, e, slot):
    h = src.shape[1] // 2
    pltpu.make_async_copy(src.at[e, :h], dst.at[slot, :h], sems.at[slot, 0]).start()
    pltpu.make_async_copy(src.at[e, h:], dst.at[slot, h:], sems.at[slot, 1]).start()


def _wait_slab(src, dst, sems, e, slot):
    h = src.shape[1] // 2
    pltpu.make_async_copy(src.at[e, :h], dst.at[slot, :h], sems.at[slot, 0]).wait()
    pltpu.make_async_copy(src.at[e, h:], dst.at[slot, h:], sems.at[slot, 1]).wait()


def _moe_kernel(xt_ref, nw_ref, gw_ref, gb_ref, m1w_hbm, m1b_ref, m2w_hbm,
                m2b_ref, out_ref, t_s, w_s, m1buf, m2buf, m1sem, m2sem):
    e = pl.program_id(0)
    T, D, I, E, K = SEQ_LEN, D_MODEL, INTERMEDIATE_SIZE, N_EXPERTS, EXPERTS_PER_TOKEN

    @pl.when(e == 0)
    def _prologue():
        for i in range(NBUF):
            _start_slab(m1w_hbm, m1buf, m1sem, i, i)
            _start_slab(m2w_hbm, m2buf, m2sem, i, i)
        xt = xt_ref[...]
        var = jnp.mean(xt * xt, axis=1, keepdims=True)
        t = xt * jax.lax.rsqrt(var + RMS_NORM_EPS) * nw_ref[...]
        t_s[...] = t
        g = jax.lax.dot_general(t, gw_ref[...], (((1,), (1,)), ((), ())),
                                preferred_element_type=jnp.float32)
        g = g + gb_ref[...]
        lane = jax.lax.broadcasted_iota(jnp.int32, (T, E), 1)
        rank = jnp.zeros((T, E), dtype=jnp.int32)
        for j in range(E):
            gj = g[:, j:j + 1]
            rank = rank + (gj > g).astype(jnp.int32)
            rank = rank + ((gj == g) & (j < lane)).astype(jnp.int32)
        sel = rank < K
        gm = jnp.where(sel, g, jnp.float32(-jnp.inf))
        mx = jnp.max(gm, axis=1, keepdims=True)
        ex = jnp.where(sel, jnp.exp(g - mx), 0.0)
        w_s[...] = ex / jnp.sum(ex, axis=1, keepdims=True)

    slot = jax.lax.rem(e, NBUF)
    _wait_slab(m1w_hbm, m1buf, m1sem, e, slot)
    _wait_slab(m2w_hbm, m2buf, m2sem, e, slot)

    t = t_s[...]
    h = jnp.dot(t, m1buf[slot], preferred_element_type=jnp.float32)
    h = h + m1b_ref[0]
    h_glu = jnp.minimum(h[:, :I], SWIGLU_LIMIT)
    h_lin = jnp.clip(h[:, I:], -SWIGLU_LIMIT, SWIGLU_LIMIT)
    act = h_glu * jax.nn.sigmoid(1.702 * h_glu) * (h_lin + 1.0)
    o = jnp.dot(act, m2buf[slot], preferred_element_type=jnp.float32)
    o = o + m2b_ref[0]

    lane_e = jax.lax.broadcasted_iota(jnp.int32, (T, E), 1)
    wcol = jnp.sum(jnp.where(lane_e == e, w_s[...], 0.0), axis=1,
                   keepdims=True)
    contrib = wcol * o

    @pl.when(e == 0)
    def _init():
        out_ref[...] = xt_ref[...] + contrib

    @pl.when(e != 0)
    def _acc():
        out_ref[...] += contrib

    @pl.when(e + NBUF < E)
    def _prefetch():
        nxt = e + NBUF
        _start_slab(m1w_hbm, m1buf, m1sem, nxt, slot)
        _start_slab(m2w_hbm, m2buf, m2sem, nxt, slot)


@jax.jit
def kernel(x, norm_weight, gate_weight, gate_bias, mlp1_weight, mlp1_bias,
           mlp2_weight, mlp2_bias):
    T, D, I, E = SEQ_LEN, D_MODEL, INTERMEDIATE_SIZE, N_EXPERTS
    xt = x.reshape(D, T).T
    out = pl.pallas_call(
        _moe_kernel,
        grid=(E,),
        in_specs=[
            pl.BlockSpec((T, D), lambda e: (0, 0)),
            pl.BlockSpec((1, D), lambda e: (0, 0)),
            pl.BlockSpec((E, D), lambda e: (0, 0)),
            pl.BlockSpec((1, E), lambda e: (0, 0)),
            pl.BlockSpec(memory_space=pl.ANY),
            pl.BlockSpec((1, 1, 2 * I), lambda e: (e, 0, 0)),
            pl.BlockSpec(memory_space=pl.ANY),
            pl.BlockSpec((1, 1, D), lambda e: (e, 0, 0)),
        ],
        out_specs=pl.BlockSpec((T, D), lambda e: (0, 0)),
        out_shape=jax.ShapeDtypeStruct((T, D), jnp.float32),
        scratch_shapes=[
            pltpu.VMEM((T, D), jnp.float32),
            pltpu.VMEM((T, E), jnp.float32),
            pltpu.VMEM((NBUF, D, 2 * I), jnp.float32),
            pltpu.VMEM((NBUF, I, D), jnp.float32),
            pltpu.SemaphoreType.DMA((NBUF, 2)),
            pltpu.SemaphoreType.DMA((NBUF, 2)),
        ],
        compiler_params=pltpu.CompilerParams(
            dimension_semantics=("arbitrary",),
        ),
    )(xt, norm_weight.reshape(1, D), gate_weight, gate_bias.reshape(1, E),
      mlp1_weight, mlp1_bias.reshape(E, 1, 2 * I), mlp2_weight,
      mlp2_bias.reshape(E, 1, D))
    return out.T.reshape(1, D, 1, T)


# speedup vs baseline: 1.1270x; 1.0483x over previous
import jax
import jax.numpy as jnp
from jax.experimental import pallas as pl
from jax.experimental.pallas import tpu as pltpu

D_MODEL = 640
INTERMEDIATE_SIZE = 640
EXPERTS_PER_TOKEN = 4
RMS_NORM_EPS = 1e-05
SWIGLU_LIMIT = 7.0
N_EXPERTS = 16
SEQ_LEN = 32
NBUF = 4


def _start_slab(src, dst, sems, e, slot):
    pltpu.make_async_copy(src.at[e], dst.at[slot], sems.at[slot]).start()


def _wait_slab(src, dst, sems, e, slot):
    pltpu.make_async_copy(src.at[e], dst.at[slot], sems.at[slot]).wait()


def _moe_kernel(xt_ref, nw_ref, gw_ref, gb_ref, m1w_hbm, m1b_ref, m2w_hbm,
                m2b_ref, out_ref, t_s, w_s, m1buf, m2buf, m1sem, m2sem):
    e = pl.program_id(0)
    T, D, I, E, K = SEQ_LEN, D_MODEL, INTERMEDIATE_SIZE, N_EXPERTS, EXPERTS_PER_TOKEN

    @pl.when(e == 0)
    def _prologue():
        for i in range(NBUF):
            _start_slab(m1w_hbm, m1buf, m1sem, i, i)
            _start_slab(m2w_hbm, m2buf, m2sem, i, i)
        xt = xt_ref[...]
        var = jnp.mean(xt * xt, axis=1, keepdims=True)
        t = xt * jax.lax.rsqrt(var + RMS_NORM_EPS) * nw_ref[...]
        t_s[...] = t
        g = jax.lax.dot_general(t, gw_ref[...], (((1,), (1,)), ((), ())),
                                preferred_element_type=jnp.float32)
        g = g + gb_ref[...]
        lane = jax.lax.broadcasted_iota(jnp.int32, (T, E), 1)
        rank = jnp.zeros((T, E), dtype=jnp.int32)
        for j in range(E):
            gj = g[:, j:j + 1]
            rank = rank + (gj > g).astype(jnp.int32)
            rank = rank + ((gj == g) & (j < lane)).astype(jnp.int32)
        sel = rank < K
        gm = jnp.where(sel, g, jnp.float32(-jnp.inf))
        mx = jnp.max(gm, axis=1, keepdims=True)
        ex = jnp.where(sel, jnp.exp(g - mx), 0.0)
        w_s[...] = ex / jnp.sum(ex, axis=1, keepdims=True)

    slot = jax.lax.rem(e, NBUF)
    _wait_slab(m1w_hbm, m1buf, m1sem, e, slot)
    _wait_slab(m2w_hbm, m2buf, m2sem, e, slot)

    t = t_s[...]
    h = jnp.dot(t, m1buf[slot], preferred_element_type=jnp.float32)
    h = h + m1b_ref[0]
    h_glu = jnp.minimum(h[:, :I], SWIGLU_LIMIT)
    h_lin = jnp.clip(h[:, I:], -SWIGLU_LIMIT, SWIGLU_LIMIT)
    act = h_glu * jax.nn.sigmoid(1.702 * h_glu) * (h_lin + 1.0)
    o = jnp.dot(act, m2buf[slot], preferred_element_type=jnp.float32)
    o = o + m2b_ref[0]

    lane_e = jax.lax.broadcasted_iota(jnp.int32, (T, E), 1)
    wcol = jnp.sum(jnp.where(lane_e == e, w_s[...], 0.0), axis=1,
                   keepdims=True)
    contrib = wcol * o

    @pl.when(e == 0)
    def _init():
        out_ref[...] = xt_ref[...] + contrib

    @pl.when(e != 0)
    def _acc():
        out_ref[...] += contrib

    @pl.when(e + NBUF < E)
    def _prefetch():
        nxt = e + NBUF
        _start_slab(m1w_hbm, m1buf, m1sem, nxt, slot)
        _start_slab(m2w_hbm, m2buf, m2sem, nxt, slot)


@jax.jit
def kernel(x, norm_weight, gate_weight, gate_bias, mlp1_weight, mlp1_bias,
           mlp2_weight, mlp2_bias):
    T, D, I, E = SEQ_LEN, D_MODEL, INTERMEDIATE_SIZE, N_EXPERTS
    xt = x.reshape(D, T).T
    out = pl.pallas_call(
        _moe_kernel,
        grid=(E,),
        in_specs=[
            pl.BlockSpec((T, D), lambda e: (0, 0)),
            pl.BlockSpec((1, D), lambda e: (0, 0)),
            pl.BlockSpec((E, D), lambda e: (0, 0)),
            pl.BlockSpec((1, E), lambda e: (0, 0)),
            pl.BlockSpec(memory_space=pl.ANY),
            pl.BlockSpec((1, 1, 2 * I), lambda e: (e, 0, 0)),
            pl.BlockSpec(memory_space=pl.ANY),
            pl.BlockSpec((1, 1, D), lambda e: (e, 0, 0)),
        ],
        out_specs=pl.BlockSpec((T, D), lambda e: (0, 0)),
        out_shape=jax.ShapeDtypeStruct((T, D), jnp.float32),
        scratch_shapes=[
            pltpu.VMEM((T, D), jnp.float32),
            pltpu.VMEM((T, E), jnp.float32),
            pltpu.VMEM((NBUF, D, 2 * I), jnp.float32),
            pltpu.VMEM((NBUF, I, D), jnp.float32),
            pltpu.SemaphoreType.DMA((NBUF,)),
            pltpu.SemaphoreType.DMA((NBUF,)),
        ],
        compiler_params=pltpu.CompilerParams(
            dimension_semantics=("arbitrary",),
        ),
    )(xt, norm_weight.reshape(1, D), gate_weight, gate_bias.reshape(1, E),
      mlp1_weight, mlp1_bias.reshape(E, 1, 2 * I), mlp2_weight,
      mlp2_bias.reshape(E, 1, D))
    return out.T.reshape(1, D, 1, T)


# speedup vs baseline: 1.2666x; 1.1239x over previous
import jax
import jax.numpy as jnp
from jax.experimental import pallas as pl
from jax.experimental.pallas import tpu as pltpu

D_MODEL = 640
INTERMEDIATE_SIZE = 640
EXPERTS_PER_TOKEN = 4
RMS_NORM_EPS = 1e-05
SWIGLU_LIMIT = 7.0
N_EXPERTS = 16
SEQ_LEN = 32
NBUF = 4


def _moe_kernel(xt_ref, nw_ref, gw_ref, gb_ref, m1w_hbm, m1b_ref, m2w_hbm,
                m2b_ref, out_ref, m1buf, m2buf, m1sem, m2sem):
    T, D, I, E, K = SEQ_LEN, D_MODEL, INTERMEDIATE_SIZE, N_EXPERTS, EXPERTS_PER_TOKEN

    for i in range(NBUF):
        pltpu.make_async_copy(m1w_hbm.at[i], m1buf.at[i], m1sem.at[i]).start()
        pltpu.make_async_copy(m2w_hbm.at[i], m2buf.at[i], m2sem.at[i]).start()

    xt = xt_ref[...]
    var = jnp.mean(xt * xt, axis=1, keepdims=True)
    t = xt * jax.lax.rsqrt(var + RMS_NORM_EPS) * nw_ref[...]
    g = jax.lax.dot_general(t, gw_ref[...], (((1,), (1,)), ((), ())),
                            preferred_element_type=jnp.float32)
    g = g + gb_ref[...]
    lane = jax.lax.broadcasted_iota(jnp.int32, (T, E), 1)
    rank = jnp.zeros((T, E), dtype=jnp.int32)
    for j in range(E):
        gj = g[:, j:j + 1]
        rank = rank + (gj > g).astype(jnp.int32)
        rank = rank + ((gj == g) & (j < lane)).astype(jnp.int32)
    sel = rank < K
    gm = jnp.where(sel, g, jnp.float32(-jnp.inf))
    mx = jnp.max(gm, axis=1, keepdims=True)
    ex = jnp.where(sel, jnp.exp(g - mx), 0.0)
    w = ex / jnp.sum(ex, axis=1, keepdims=True)

    acc = xt
    for e in range(E):
        slot = e % NBUF
        pltpu.make_async_copy(m1w_hbm.at[e], m1buf.at[slot], m1sem.at[slot]).wait()
        h = jnp.dot(t, m1buf[slot], preferred_element_type=jnp.float32)
        h = h + m1b_ref[e:e + 1, :]
        h_glu = jnp.minimum(h[:, :I], SWIGLU_LIMIT)
        h_lin = jnp.clip(h[:, I:], -SWIGLU_LIMIT, SWIGLU_LIMIT)
        act = h_glu * jax.nn.sigmoid(1.702 * h_glu) * (h_lin + 1.0)
        pltpu.make_async_copy(m2w_hbm.at[e], m2buf.at[slot], m2sem.at[slot]).wait()
        o = jnp.dot(act, m2buf[slot], preferred_element_type=jnp.float32)
        o = o + m2b_ref[e:e + 1, :]
        acc = acc + w[:, e:e + 1] * o
        nxt = e + NBUF
        if nxt < E:
            pltpu.make_async_copy(m1w_hbm.at[nxt], m1buf.at[slot], m1sem.at[slot]).start()
            pltpu.make_async_copy(m2w_hbm.at[nxt], m2buf.at[slot], m2sem.at[slot]).start()
    out_ref[...] = acc


@jax.jit
def kernel(x, norm_weight, gate_weight, gate_bias, mlp1_weight, mlp1_bias,
           mlp2_weight, mlp2_bias):
    T, D, I, E = SEQ_LEN, D_MODEL, INTERMEDIATE_SIZE, N_EXPERTS
    xt = x.reshape(D, T).T
    out = pl.pallas_call(
        _moe_kernel,
        in_specs=[
            pl.BlockSpec((T, D), lambda: (0, 0)),
            pl.BlockSpec((1, D), lambda: (0, 0)),
            pl.BlockSpec((E, D), lambda: (0, 0)),
            pl.BlockSpec((1, E), lambda: (0, 0)),
            pl.BlockSpec(memory_space=pl.ANY),
            pl.BlockSpec((E, 2 * I), lambda: (0, 0)),
            pl.BlockSpec(memory_space=pl.ANY),
            pl.BlockSpec((E, D), lambda: (0, 0)),
        ],
        out_specs=pl.BlockSpec((T, D), lambda: (0, 0)),
        out_shape=jax.ShapeDtypeStruct((T, D), jnp.float32),
        scratch_shapes=[
            pltpu.VMEM((NBUF, D, 2 * I), jnp.float32),
            pltpu.VMEM((NBUF, I, D), jnp.float32),
            pltpu.SemaphoreType.DMA((NBUF,)),
            pltpu.SemaphoreType.DMA((NBUF,)),
        ],
    )(xt, norm_weight.reshape(1, D), gate_weight, gate_bias.reshape(1, E),
      mlp1_weight, mlp1_bias, mlp2_weight, mlp2_bias)
    return out.T.reshape(1, D, 1, T)
